# baseline (speedup 1.0000x reference)
import jax
import jax.numpy as jnp
from jax.experimental import pallas as pl


def _aidw_body(src_ref, tar_ref, mask_ref, feat_ref, lin_ref, out_ref):
    G, TL, S = feat_ref.shape
    diff = src_ref[...] - tar_ref[...]
    d2 = jnp.sum(diff * diff, axis=2, keepdims=True)
    inv = jnp.swapaxes(1.0 / d2, 1, 2)
    sc = jnp.where(mask_ref[...] != 0, inv, 0.0)
    w = sc / jnp.sum(sc, axis=2, keepdims=True)
    scaled = feat_ref[...] * w
    out = jnp.dot(scaled.reshape(G * TL, S), lin_ref[...],
                  preferred_element_type=jnp.float32)
    out_ref[...] = out.reshape(G, TL, out.shape[1])


def kernel(features, src_locs, tar_loc, src_masks, linear):
    B, L, S = features.shape
    O = linear.shape[1]
    tar_r = tar_loc.reshape(B, 1, 2)
    mask_r = src_masks.reshape(B, 1, S)

    G, TL = 16, 1024
    return pl.pallas_call(
        _aidw_body,
        grid=(B // G, L // TL),
        in_specs=[
            pl.BlockSpec((G, S, 2), lambda b, l: (b, 0, 0)),
            pl.BlockSpec((G, 1, 2), lambda b, l: (b, 0, 0)),
            pl.BlockSpec((G, 1, S), lambda b, l: (b, 0, 0)),
            pl.BlockSpec((G, TL, S), lambda b, l: (b, l, 0)),
            pl.BlockSpec((S, O), lambda b, l: (0, 0)),
        ],
        out_specs=pl.BlockSpec((G, TL, O), lambda b, l: (b, l, 0)),
        out_shape=jax.ShapeDtypeStruct((B, L, O), jnp.float32),
    )(src_locs, tar_r, mask_r, features, linear)


# speedup vs baseline: 1.0466x; 1.0466x over previous
import jax
import jax.numpy as jnp
from jax.experimental import pallas as pl
from jax.experimental.pallas import tpu as pltpu


def _aidw_body(src_ref, tar_ref, mask_ref, feat_ref, lin_ref, out_ref):
    G, TL, S = feat_ref.shape
    diff = src_ref[...] - tar_ref[...]
    d2 = jnp.sum(diff * diff, axis=1, keepdims=True)
    sc = jnp.where(mask_ref[...] != 0.0, 1.0 / d2, 0.0)
    w = sc / jnp.sum(sc, axis=2, keepdims=True)
    scaled = feat_ref[...] * w
    out = jnp.dot(scaled.reshape(G * TL, S), lin_ref[...],
                  preferred_element_type=jnp.float32)
    out_ref[...] = out.reshape(G, TL, out.shape[1])


def kernel(features, src_locs, tar_loc, src_masks, linear):
    B, L, S = features.shape
    O = linear.shape[1]
    src_t = jnp.transpose(src_locs, (0, 2, 1))
    tar_b = tar_loc[:, :, None]
    mask_f = src_masks.astype(jnp.float32)[:, None, :]

    G, TL = 8, 2048
    return pl.pallas_call(
        _aidw_body,
        grid=(B // G, L // TL),
        in_specs=[
            pl.BlockSpec((G, 2, S), lambda b, l: (b, 0, 0)),
            pl.BlockSpec((G, 2, 1), lambda b, l: (b, 0, 0)),
            pl.BlockSpec((G, 1, S), lambda b, l: (b, 0, 0)),
            pl.BlockSpec((G, TL, S), lambda b, l: (b, l, 0)),
            pl.BlockSpec((S, O), lambda b, l: (0, 0)),
        ],
        out_specs=pl.BlockSpec((G, TL, O), lambda b, l: (b, l, 0)),
        out_shape=jax.ShapeDtypeStruct((B, L, O), jnp.float32),
        compiler_params=pltpu.CompilerParams(
            dimension_semantics=("parallel", "parallel")),
    )(src_t, tar_b, mask_f, features, linear)
